# Initial kernel scaffold; baseline (speedup 1.0000x reference)
import jax
import jax.numpy as jnp
from jax.experimental import pallas as pl


def kernel(emb_event, emb_entity, W, b, x_id, pos_id, neg_id):
    raise NotImplementedError("write your pallas kernel here")


# speedup vs baseline: 1.7317x; 1.7317x over previous
import functools

import jax
import jax.numpy as jnp
from jax import lax
from jax.experimental import pallas as pl
from jax.experimental.pallas import tpu as pltpu
from jax.experimental.pallas import tpu_sc as plsc

NEMB = 128
NC = 2
NS = 16
NW = NC * NS
GCHUNK = 128


def _make_gather3(batch):
    b_per_w = batch // NW
    n_chunks = b_per_w // GCHUNK
    mesh = plsc.VectorSubcoreMesh(core_axis_name="c", subcore_axis_name="s")
    out_row = jax.ShapeDtypeStruct((batch, NEMB), jnp.float32)

    @functools.partial(
        pl.kernel,
        out_type=(out_row, out_row, out_row),
        mesh=mesh,
        scratch_types=[
            pltpu.VMEM((n_chunks, GCHUNK), jnp.int32),
            pltpu.VMEM((GCHUNK, NEMB), jnp.float32),
            pltpu.VMEM((GCHUNK, NEMB), jnp.float32),
            pltpu.SemaphoreType.DMA,
            pltpu.SemaphoreType.DMA,
        ],
    )
    def gather3(ev_hbm, en_hbm, xid_hbm, pid_hbm, nid_hbm,
                out_x, out_p, out_n,
                idx_v, rows_a, rows_b, gsem_a, gsem_b):
        wid = lax.axis_index("s") * NC + lax.axis_index("c")
        base = wid * b_per_w
        rows = (rows_a, rows_b)
        gsems = (gsem_a, gsem_b)

        jobs = ((ev_hbm, xid_hbm, out_x),
                (en_hbm, pid_hbm, out_p),
                (en_hbm, nid_hbm, out_n))

        for table, idx_hbm, out in jobs:
            pltpu.sync_copy(idx_hbm.at[pl.ds(wid * n_chunks, n_chunks)],
                            idx_v)
            handles = [None] * n_chunks
            handles[0] = pltpu.make_async_copy(
                table.at[idx_v.at[0]], rows[0], gsems[0])
            handles[0].start()
            for c in range(n_chunks):
                nxt = c + 1
                if nxt < n_chunks:
                    handles[nxt] = pltpu.make_async_copy(
                        table.at[idx_v.at[nxt]], rows[nxt % 2],
                        gsems[nxt % 2])
                    handles[nxt].start()
                handles[c].wait()
                pltpu.sync_copy(rows[c % 2],
                                out.at[pl.ds(base + c * GCHUNK, GCHUNK)])

    return gather3


def _make_bilinear_loss(batch):
    tb = 512
    grid = (batch // tb,)
    inv = 0.5 / batch

    def body(x_ref, p_ref, n_ref, w_ref, b_ref, out_ref):
        i = pl.program_id(0)
        u = jnp.dot(x_ref[...], w_ref[...],
                    preferred_element_type=jnp.float32)
        bias = b_ref[0]
        d1 = jnp.sum(u * p_ref[...], axis=1, keepdims=True) + bias
        d2 = jnp.sum(u * n_ref[...], axis=1, keepdims=True) + bias
        l1 = jnp.maximum(d1, 0.0) - d1 + jnp.log(1.0 + jnp.exp(-jnp.abs(d1)))
        l2 = jnp.maximum(d2, 0.0) + jnp.log(1.0 + jnp.exp(-jnp.abs(d2)))
        part = jnp.sum(l1) + jnp.sum(l2)

        @pl.when(i == 0)
        def _():
            out_ref[0] = 0.0

        out_ref[0] += part

        @pl.when(i == grid[0] - 1)
        def _():
            out_ref[0] = out_ref[0] * inv

    return pl.pallas_call(
        body,
        grid=grid,
        in_specs=[
            pl.BlockSpec((tb, NEMB), lambda i: (i, 0)),
            pl.BlockSpec((tb, NEMB), lambda i: (i, 0)),
            pl.BlockSpec((tb, NEMB), lambda i: (i, 0)),
            pl.BlockSpec((NEMB, NEMB), lambda i: (0, 0)),
            pl.BlockSpec(memory_space=pltpu.SMEM),
        ],
        out_specs=pl.BlockSpec(memory_space=pltpu.SMEM),
        out_shape=jax.ShapeDtypeStruct((1,), jnp.float32),
    )


def kernel(emb_event, emb_entity, W, b, x_id, pos_id, neg_id):
    batch = x_id.shape[0]
    gather3 = _make_gather3(batch)
    x_g, p_g, n_g = gather3(
        emb_event, emb_entity,
        x_id.astype(jnp.int32).reshape(batch // GCHUNK, GCHUNK),
        pos_id.astype(jnp.int32).reshape(batch // GCHUNK, GCHUNK),
        neg_id.astype(jnp.int32).reshape(batch // GCHUNK, GCHUNK))
    loss = _make_bilinear_loss(batch)(x_g, p_g, n_g, W[0], b)
    return loss[0]


# speedup vs baseline: 1.8030x; 1.0412x over previous
import functools

import jax
import jax.numpy as jnp
from jax import lax
from jax.experimental import pallas as pl
from jax.experimental.pallas import tpu as pltpu
from jax.experimental.pallas import tpu_sc as plsc

NEMB = 128
NC = 2
NS = 16
NW = NC * NS
GCHUNK = 128


def _make_gather3(batch):
    b_per_w = batch // NW
    n_chunks = b_per_w // GCHUNK
    mesh = plsc.VectorSubcoreMesh(core_axis_name="c", subcore_axis_name="s")
    out_row = jax.ShapeDtypeStruct((batch, NEMB), jnp.float32)

    @functools.partial(
        pl.kernel,
        out_type=(out_row, out_row, out_row),
        mesh=mesh,
        scratch_types=[
            pltpu.VMEM((n_chunks, GCHUNK), jnp.int32),
            pltpu.VMEM((GCHUNK, NEMB), jnp.float32),
            pltpu.VMEM((GCHUNK, NEMB), jnp.float32),
            pltpu.SemaphoreType.DMA,
            pltpu.SemaphoreType.DMA,
        ],
    )
    def gather3(ev_hbm, en_hbm, xid_hbm, pid_hbm, nid_hbm,
                out_x, out_p, out_n,
                idx_v, rows_a, rows_b, gsem_a, gsem_b):
        wid = lax.axis_index("s") * NC + lax.axis_index("c")
        base = wid * b_per_w
        rows = (rows_a, rows_b)
        gsems = (gsem_a, gsem_b)

        jobs = ((ev_hbm, xid_hbm, out_x),
                (en_hbm, pid_hbm, out_p),
                (en_hbm, nid_hbm, out_n))

        for table, idx_hbm, out in jobs:
            pltpu.sync_copy(idx_hbm.at[pl.ds(wid * n_chunks, n_chunks)],
                            idx_v)
            handles = [None] * n_chunks
            handles[0] = pltpu.make_async_copy(
                table.at[idx_v.at[0]], rows[0], gsems[0])
            handles[0].start()
            for c in range(n_chunks):
                nxt = c + 1
                if nxt < n_chunks:
                    handles[nxt] = pltpu.make_async_copy(
                        table.at[idx_v.at[nxt]], rows[nxt % 2],
                        gsems[nxt % 2])
                    handles[nxt].start()
                handles[c].wait()
                pltpu.sync_copy(rows[c % 2],
                                out.at[pl.ds(base + c * GCHUNK, GCHUNK)])

    return gather3


def _make_bilinear_loss(batch):
    tb = 512
    grid = (batch // tb,)

    def body(x_ref, p_ref, n_ref, w_ref, b_ref, out_ref):
        i = pl.program_id(0)
        u = jnp.dot(x_ref[...], w_ref[...],
                    preferred_element_type=jnp.float32)
        bias = b_ref[0]
        d1 = jnp.sum(u * p_ref[...], axis=1, keepdims=True) + bias
        d2 = jnp.sum(u * n_ref[...], axis=1, keepdims=True) + bias
        l1 = jnp.maximum(d1, 0.0) - d1 + jnp.log(1.0 + jnp.exp(-jnp.abs(d1)))
        l2 = jnp.maximum(d2, 0.0) + jnp.log(1.0 + jnp.exp(-jnp.abs(d2)))
        part = jnp.sum(l1) + jnp.sum(l2)

        @pl.when(i == 0)
        def _():
            out_ref[0] = 0.0

        out_ref[0] += part

    return pl.pallas_call(
        body,
        grid=grid,
        in_specs=[
            pl.BlockSpec((tb, NEMB), lambda i: (i, 0)),
            pl.BlockSpec((tb, NEMB), lambda i: (i, 0)),
            pl.BlockSpec((tb, NEMB), lambda i: (i, 0)),
            pl.BlockSpec((NEMB, NEMB), lambda i: (0, 0)),
            pl.BlockSpec(memory_space=pltpu.SMEM),
        ],
        out_specs=pl.BlockSpec(memory_space=pltpu.SMEM),
        out_shape=jax.ShapeDtypeStruct((1,), jnp.float32),
    )


def kernel(emb_event, emb_entity, W, b, x_id, pos_id, neg_id):
    batch = x_id.shape[0]
    nsplit = 2
    chunk = batch // nsplit
    gather3 = _make_gather3(chunk)
    tc_loss = _make_bilinear_loss(chunk)
    xi = x_id.astype(jnp.int32).reshape(nsplit, chunk // GCHUNK, GCHUNK)
    pi = pos_id.astype(jnp.int32).reshape(nsplit, chunk // GCHUNK, GCHUNK)
    ni = neg_id.astype(jnp.int32).reshape(nsplit, chunk // GCHUNK, GCHUNK)
    w0 = W[0]
    total = None
    for k in range(nsplit):
        x_g, p_g, n_g = gather3(emb_event, emb_entity, xi[k], pi[k], ni[k])
        part = _make_bilinear_loss(chunk)(x_g, p_g, n_g, w0, b)[0]
        total = part if total is None else total + part
    return total * (0.5 / batch)


# speedup vs baseline: 1.8277x; 1.0137x over previous
import functools

import jax
import jax.numpy as jnp
from jax import lax
from jax.experimental import pallas as pl
from jax.experimental.pallas import tpu as pltpu
from jax.experimental.pallas import tpu_sc as plsc

NEMB = 128
NC = 2
NS = 16
NW = NC * NS
GCHUNK = 128


def _make_gather3(batch):
    b_per_w = batch // NW
    n_chunks = b_per_w // GCHUNK
    mesh = plsc.VectorSubcoreMesh(core_axis_name="c", subcore_axis_name="s")
    out_row = jax.ShapeDtypeStruct((batch, NEMB), jnp.float32)

    @functools.partial(
        pl.kernel,
        out_type=(out_row, out_row, out_row),
        mesh=mesh,
        scratch_types=[
            pltpu.VMEM((n_chunks, GCHUNK), jnp.int32),
            pltpu.VMEM((GCHUNK, NEMB), jnp.float32),
            pltpu.VMEM((GCHUNK, NEMB), jnp.float32),
            pltpu.SemaphoreType.DMA,
            pltpu.SemaphoreType.DMA,
        ],
    )
    def gather3(ev_hbm, en_hbm, xid_hbm, pid_hbm, nid_hbm,
                out_x, out_p, out_n,
                idx_v, rows_a, rows_b, gsem_a, gsem_b):
        wid = lax.axis_index("s") * NC + lax.axis_index("c")
        base = wid * b_per_w
        rows = (rows_a, rows_b)
        gsems = (gsem_a, gsem_b)

        jobs = ((ev_hbm, xid_hbm, out_x),
                (en_hbm, pid_hbm, out_p),
                (en_hbm, nid_hbm, out_n))

        for table, idx_hbm, out in jobs:
            pltpu.sync_copy(idx_hbm.at[pl.ds(wid * n_chunks, n_chunks)],
                            idx_v)
            handles = [None] * n_chunks
            handles[0] = pltpu.make_async_copy(
                table.at[idx_v.at[0]], rows[0], gsems[0])
            handles[0].start()
            for c in range(n_chunks):
                nxt = c + 1
                if nxt < n_chunks:
                    handles[nxt] = pltpu.make_async_copy(
                        table.at[idx_v.at[nxt]], rows[nxt % 2],
                        gsems[nxt % 2])
                    handles[nxt].start()
                handles[c].wait()
                pltpu.sync_copy(rows[c % 2],
                                out.at[pl.ds(base + c * GCHUNK, GCHUNK)])

    return gather3


def _make_bilinear_loss(batch):
    tb = 512
    grid = (batch // tb,)

    def body(x_ref, p_ref, n_ref, w_ref, b_ref, out_ref):
        i = pl.program_id(0)
        u = jnp.dot(x_ref[...], w_ref[...],
                    preferred_element_type=jnp.float32)
        bias = b_ref[0]
        rr = lax.broadcasted_iota(jnp.int32, (NEMB, NEMB), 0)
        cc = lax.broadcasted_iota(jnp.int32, (NEMB, NEMB), 1)
        eye = (rr == cc).astype(jnp.float32)
        d1s, d2s = [], []
        for c in range(tb // NEMB):
            uc = lax.slice(u, (c * NEMB, 0), ((c + 1) * NEMB, NEMB))
            pc = p_ref[pl.ds(c * NEMB, NEMB), :]
            nc = n_ref[pl.ds(c * NEMB, NEMB), :]
            m1 = lax.dot_general(uc, pc, (((1,), (1,)), ((), ())),
                                 preferred_element_type=jnp.float32)
            m2 = lax.dot_general(uc, nc, (((1,), (1,)), ((), ())),
                                 preferred_element_type=jnp.float32)
            d1s.append(jnp.sum(m1 * eye, axis=0, keepdims=True))
            d2s.append(jnp.sum(m2 * eye, axis=0, keepdims=True))
        d1 = jnp.concatenate(d1s, axis=0) + bias
        d2 = jnp.concatenate(d2s, axis=0) + bias
        l1 = jnp.maximum(d1, 0.0) - d1 + jnp.log(1.0 + jnp.exp(-jnp.abs(d1)))
        l2 = jnp.maximum(d2, 0.0) + jnp.log(1.0 + jnp.exp(-jnp.abs(d2)))
        part = jnp.sum(l1 + l2)

        @pl.when(i == 0)
        def _():
            out_ref[0] = 0.0

        out_ref[0] += part

    return pl.pallas_call(
        body,
        grid=grid,
        in_specs=[
            pl.BlockSpec((tb, NEMB), lambda i: (i, 0)),
            pl.BlockSpec((tb, NEMB), lambda i: (i, 0)),
            pl.BlockSpec((tb, NEMB), lambda i: (i, 0)),
            pl.BlockSpec((NEMB, NEMB), lambda i: (0, 0)),
            pl.BlockSpec(memory_space=pltpu.SMEM),
        ],
        out_specs=pl.BlockSpec(memory_space=pltpu.SMEM),
        out_shape=jax.ShapeDtypeStruct((1,), jnp.float32),
    )


def kernel(emb_event, emb_entity, W, b, x_id, pos_id, neg_id):
    batch = x_id.shape[0]
    nsplit = 2
    chunk = batch // nsplit
    gather3 = _make_gather3(chunk)
    tc_loss = _make_bilinear_loss(chunk)
    xi = x_id.astype(jnp.int32).reshape(nsplit, chunk // GCHUNK, GCHUNK)
    pi = pos_id.astype(jnp.int32).reshape(nsplit, chunk // GCHUNK, GCHUNK)
    ni = neg_id.astype(jnp.int32).reshape(nsplit, chunk // GCHUNK, GCHUNK)
    w0 = W[0]
    total = None
    for k in range(nsplit):
        x_g, p_g, n_g = gather3(emb_event, emb_entity, xi[k], pi[k], ni[k])
        part = _make_bilinear_loss(chunk)(x_g, p_g, n_g, w0, b)[0]
        total = part if total is None else total + part
    return total * (0.5 / batch)


# speedup vs baseline: 2.0681x; 1.1315x over previous
import functools

import jax
import jax.numpy as jnp
from jax import lax
from jax.experimental import pallas as pl
from jax.experimental.pallas import tpu as pltpu
from jax.experimental.pallas import tpu_sc as plsc

NEMB = 128
NC = 2
NS = 16
NW = NC * NS
GCHUNK = 128


def _make_gather3(chunk, k):
    b_per_w = chunk // NW
    n_chunks = b_per_w // GCHUNK
    mesh = plsc.VectorSubcoreMesh(core_axis_name="c", subcore_axis_name="s")
    out_row = jax.ShapeDtypeStruct((chunk, NEMB), jnp.float32)
    row0 = k * (chunk // GCHUNK)

    @functools.partial(
        pl.kernel,
        out_type=(out_row, out_row, out_row),
        mesh=mesh,
        scratch_types=[
            pltpu.VMEM((n_chunks, GCHUNK), jnp.int32),
            pltpu.VMEM((GCHUNK, NEMB), jnp.float32),
            pltpu.VMEM((GCHUNK, NEMB), jnp.float32),
            pltpu.SemaphoreType.DMA,
            pltpu.SemaphoreType.DMA,
        ],
    )
    def gather3(ev_hbm, en_hbm, xid_hbm, pid_hbm, nid_hbm,
                out_x, out_p, out_n,
                idx_v, rows_a, rows_b, gsem_a, gsem_b):
        wid = lax.axis_index("s") * NC + lax.axis_index("c")
        base = wid * b_per_w
        rows = (rows_a, rows_b)
        gsems = (gsem_a, gsem_b)

        jobs = ((ev_hbm, xid_hbm, out_x),
                (en_hbm, pid_hbm, out_p),
                (en_hbm, nid_hbm, out_n))

        for table, idx_hbm, out in jobs:
            pltpu.sync_copy(
                idx_hbm.at[pl.ds(row0 + wid * n_chunks, n_chunks)], idx_v)
            handles = [None] * n_chunks
            handles[0] = pltpu.make_async_copy(
                table.at[idx_v.at[0]], rows[0], gsems[0])
            handles[0].start()
            for c in range(n_chunks):
                nxt = c + 1
                if nxt < n_chunks:
                    handles[nxt] = pltpu.make_async_copy(
                        table.at[idx_v.at[nxt]], rows[nxt % 2],
                        gsems[nxt % 2])
                    handles[nxt].start()
                handles[c].wait()
                pltpu.sync_copy(rows[c % 2],
                                out.at[pl.ds(base + c * GCHUNK, GCHUNK)])

    return gather3


def _make_bilinear_loss(batch):
    tb = 2048
    grid = (batch // tb,)

    def body(x_ref, p_ref, n_ref, w_ref, b_ref, out_ref):
        i = pl.program_id(0)
        u = jnp.dot(x_ref[...], w_ref[...],
                    preferred_element_type=jnp.float32)
        bias = b_ref[0]
        rr = lax.broadcasted_iota(jnp.int32, (NEMB, NEMB), 0)
        cc = lax.broadcasted_iota(jnp.int32, (NEMB, NEMB), 1)
        eye = (rr == cc).astype(jnp.float32)
        d1s, d2s = [], []
        for c in range(tb // NEMB):
            uc = lax.slice(u, (c * NEMB, 0), ((c + 1) * NEMB, NEMB))
            pc = p_ref[pl.ds(c * NEMB, NEMB), :]
            nc = n_ref[pl.ds(c * NEMB, NEMB), :]
            m1 = lax.dot_general(uc, pc, (((1,), (1,)), ((), ())),
                                 preferred_element_type=jnp.float32)
            m2 = lax.dot_general(uc, nc, (((1,), (1,)), ((), ())),
                                 preferred_element_type=jnp.float32)
            d1s.append(jnp.sum(m1 * eye, axis=0, keepdims=True))
            d2s.append(jnp.sum(m2 * eye, axis=0, keepdims=True))
        d1 = jnp.concatenate(d1s, axis=0) + bias
        d2 = jnp.concatenate(d2s, axis=0) + bias
        l1 = jnp.maximum(d1, 0.0) - d1 + jnp.log(1.0 + jnp.exp(-jnp.abs(d1)))
        l2 = jnp.maximum(d2, 0.0) + jnp.log(1.0 + jnp.exp(-jnp.abs(d2)))
        part = jnp.sum(l1 + l2)

        @pl.when(i == 0)
        def _():
            out_ref[0] = 0.0

        out_ref[0] += part

    return pl.pallas_call(
        body,
        grid=grid,
        in_specs=[
            pl.BlockSpec((tb, NEMB), lambda i: (i, 0)),
            pl.BlockSpec((tb, NEMB), lambda i: (i, 0)),
            pl.BlockSpec((tb, NEMB), lambda i: (i, 0)),
            pl.BlockSpec((NEMB, NEMB), lambda i: (0, 0)),
            pl.BlockSpec(memory_space=pltpu.SMEM),
        ],
        out_specs=pl.BlockSpec(memory_space=pltpu.SMEM),
        out_shape=jax.ShapeDtypeStruct((1,), jnp.float32),
    )


def kernel(emb_event, emb_entity, W, b, x_id, pos_id, neg_id):
    batch = x_id.shape[0]
    nsplit = 2
    chunk = batch // nsplit
    tc_loss = _make_bilinear_loss(chunk)
    xi = x_id.astype(jnp.int32).reshape(batch // GCHUNK, GCHUNK)
    pi = pos_id.astype(jnp.int32).reshape(batch // GCHUNK, GCHUNK)
    ni = neg_id.astype(jnp.int32).reshape(batch // GCHUNK, GCHUNK)
    w0 = W[0]
    total = None
    for k in range(nsplit):
        x_g, p_g, n_g = _make_gather3(chunk, k)(
            emb_event, emb_entity, xi, pi, ni)
        part = tc_loss(x_g, p_g, n_g, w0, b)[0]
        total = part if total is None else total + part
    return total * (0.5 / batch)
